# baseline (speedup 1.0000x reference)
import functools

import jax
import jax.numpy as jnp
from jax.experimental import pallas as pl
from jax.experimental.pallas import tpu as pltpu

_T = 256
_C = 128
_L = 128
_F = 6
_ENC_DIL = (1, 2, 4, 8)
_DEC_DIL = (8, 4, 2, 1, 1)
_ENC_LN = (True, True, True, True)
_DEC_LN = (True, True, True, True, False)
_LN_EPS = 1e-5
_SLOPE = 0.1
_MAXD = 16
_FLAT = _T * _C


def _conv_layers(buf, w_ref, b_ref, g_ref, beta_ref, bh, dilations, apply_ln,
                 ln_waits=None):
    nl = len(dilations)
    nh = 2 if bh % 2 == 0 else 1
    hh = bh // nh
    ln_idx = 0
    ys = [None] * nh
    for l in range(nl):
        d = dilations[l]
        for h in range(nh):
            b0 = h * hh
            y = None
            for tap in range(3):
                base = _MAXD + (tap - 1) * d
                lhs = buf[b0:b0 + hh, base:base + _T, :].reshape(hh * _T, _C)
                p = jnp.dot(lhs, w_ref[l, tap * _C:(tap + 1) * _C, :],
                            preferred_element_type=jnp.float32)
                y = p if y is None else y + p
            ys[h] = y.reshape(hh, _T, _C) + b_ref[l]
        if apply_ln[l] and ln_waits is not None and ln_idx in ln_waits:
            for cp in ln_waits[ln_idx]:
                cp.wait()
        for h in range(nh):
            y = ys[h]
            if apply_ln[l]:
                y = jnp.maximum(y, _SLOPE * y)
                mu = jnp.mean(y, axis=(1, 2), keepdims=True)
                msq = jnp.mean(y * y, axis=(1, 2), keepdims=True)
                var = jnp.maximum(msq - mu * mu, 0.0)
                y = (y - mu) * jax.lax.rsqrt(var + _LN_EPS)
                y = y * g_ref[ln_idx] + beta_ref[ln_idx]
                ys[h] = y
            if l + 1 < nl:
                buf[h * hh:h * hh + hh, _MAXD:_MAXD + _T, :] = \
                    y.astype(jnp.bfloat16)
        if apply_ln[l]:
            ln_idx += 1
    return ys[0] if nh == 1 else jnp.concatenate(ys, axis=0)


def _mega_body(x_ref, eps_ref, enc_w_ref, enc_b_ref, enc_g_hbm, enc_beta_hbm,
               fc_w_hbm, fc_b_ref, dec_in_w_hbm, dec_in_b_ref,
               dec_w_hbm, dec_b_ref, dec_g_hbm, dec_beta_hbm,
               recon_ref, mean_ref, logvar_ref,
               buf, enc_g_v, enc_beta_v, fc_w_v, dec_in_w_v, dec_w_v,
               dec_g_v, dec_beta_v, sems,
               *, bh):
    nfc = 4
    fck = _FLAT // nfc
    ndi = 4
    dic = _FLAT // ndi
    cp_eg = pltpu.make_async_copy(enc_g_hbm, enc_g_v, sems.at[0])
    cp_eb = pltpu.make_async_copy(enc_beta_hbm, enc_beta_v, sems.at[1])
    cp_fc = [pltpu.make_async_copy(fc_w_hbm.at[k * fck:(k + 1) * fck, :],
                                   fc_w_v.at[k * fck:(k + 1) * fck, :],
                                   sems.at[2 + k])
             for k in range(nfc)]
    cp_di = [pltpu.make_async_copy(dec_in_w_hbm.at[:, k * dic:(k + 1) * dic],
                                   dec_in_w_v.at[:, k * dic:(k + 1) * dic],
                                   sems.at[2 + nfc + k])
             for k in range(ndi)]
    cp_dw = pltpu.make_async_copy(dec_w_hbm, dec_w_v, sems.at[2 + nfc + ndi])
    cp_dg = [pltpu.make_async_copy(dec_g_hbm.at[l], dec_g_v.at[l],
                                   sems.at[3 + nfc + ndi + 2 * l])
             for l in range(4)]
    cp_db = [pltpu.make_async_copy(dec_beta_hbm.at[l], dec_beta_v.at[l],
                                   sems.at[4 + nfc + ndi + 2 * l])
             for l in range(4)]
    cp_eg.start()
    cp_eb.start()
    for cp in cp_fc:
        cp.start()
    for cp in cp_di:
        cp.start()
    cp_dw.start()
    for cpg, cpb in zip(cp_dg, cp_db):
        cpg.start()
        cpb.start()

    buf[...] = jnp.zeros(buf.shape, jnp.bfloat16)
    buf[:, _MAXD:_MAXD + _T, 0:_F] = x_ref[...].astype(jnp.bfloat16)
    y = _conv_layers(buf, enc_w_ref, enc_b_ref, enc_g_v, enc_beta_v,
                     bh, _ENC_DIL, _ENC_LN,
                     ln_waits={0: [cp_eg, cp_eb]})

    flat = y.astype(jnp.bfloat16).reshape(bh, _FLAT)
    y2 = None
    for k in range(nfc):
        cp_fc[k].wait()
        p = jnp.dot(flat[:, k * fck:(k + 1) * fck],
                    fc_w_v[k * fck:(k + 1) * fck, :],
                    preferred_element_type=jnp.float32)
        y2 = p if y2 is None else y2 + p
    y2 = y2 + fc_b_ref[...]
    mean = y2[:, 0:_L]
    logvar = y2[:, _L:]
    mean_ref[...] = mean
    logvar_ref[...] = logvar

    z = (mean + eps_ref[...] * jnp.exp(0.5 * logvar)).astype(jnp.bfloat16)
    step = 2048
    ts = step // _C
    for j in range(0, _FLAT, step):
        if j % dic == 0:
            cp_di[j // dic].wait()
        hj = jnp.dot(z, dec_in_w_v[:, j:j + step],
                     preferred_element_type=jnp.float32) + \
            dec_in_b_ref[:, j:j + step]
        t0 = _MAXD + j // _C
        buf[:, t0:t0 + ts, :] = hj.astype(jnp.bfloat16).reshape(bh, ts, _C)

    cp_dw.wait()
    y = _conv_layers(buf, dec_w_v, dec_b_ref, dec_g_v, dec_beta_v,
                     bh, _DEC_DIL, _DEC_LN,
                     ln_waits={l: [cp_dg[l], cp_db[l]] for l in range(4)})
    recon_ref[...] = y[:, :, 0:_F]


def _forward_one_device(x, eps, enc_w, enc_b, enc_g, enc_beta, fc_w, fc_b,
                        dec_in_w, dec_in_b, dec_w, dec_b, dec_g, dec_beta):
    B = x.shape[0]
    body = functools.partial(_mega_body, bh=B)
    any_spec = pl.BlockSpec(memory_space=pl.ANY)
    recon, mean, logvar = pl.pallas_call(
        body,
        out_shape=(jax.ShapeDtypeStruct((B, _T, _F), jnp.float32),
                   jax.ShapeDtypeStruct((B, _L), jnp.float32),
                   jax.ShapeDtypeStruct((B, _L), jnp.float32)),
        in_specs=[
            pl.BlockSpec(x.shape, lambda: (0, 0, 0)),
            pl.BlockSpec(eps.shape, lambda: (0, 0)),
            pl.BlockSpec(enc_w.shape, lambda: (0, 0, 0)),
            pl.BlockSpec(enc_b.shape, lambda: (0, 0, 0)),
            any_spec,
            any_spec,
            any_spec,
            pl.BlockSpec(fc_b.shape, lambda: (0, 0)),
            any_spec,
            pl.BlockSpec(dec_in_b.shape, lambda: (0, 0)),
            any_spec,
            pl.BlockSpec(dec_b.shape, lambda: (0, 0, 0)),
            any_spec,
            any_spec,
        ],
        out_specs=(
            pl.BlockSpec((B, _T, _F), lambda: (0, 0, 0)),
            pl.BlockSpec((B, _L), lambda: (0, 0)),
            pl.BlockSpec((B, _L), lambda: (0, 0)),
        ),
        scratch_shapes=[
            pltpu.VMEM((B, _T + 2 * _MAXD, _C), jnp.bfloat16),
            pltpu.VMEM(enc_g.shape, enc_g.dtype),
            pltpu.VMEM(enc_beta.shape, enc_beta.dtype),
            pltpu.VMEM(fc_w.shape, fc_w.dtype),
            pltpu.VMEM(dec_in_w.shape, dec_in_w.dtype),
            pltpu.VMEM(dec_w.shape, dec_w.dtype),
            pltpu.VMEM(dec_g.shape, dec_g.dtype),
            pltpu.VMEM(dec_beta.shape, dec_beta.dtype),
            pltpu.SemaphoreType.DMA((19,)),
        ],
    )(x, eps, enc_w, enc_b, enc_g, enc_beta, fc_w, fc_b,
      dec_in_w, dec_in_b, dec_w, dec_b, dec_g, dec_beta)
    return recon, mean, logvar


def kernel(x, eps, enc_w, enc_b, enc_g, enc_beta, fc_w, fc_b,
           dec_in_w, dec_in_b, dec_w, dec_b, dec_g, dec_beta):
    return _forward_one_device(x, eps, enc_w, enc_b, enc_g, enc_beta,
                               fc_w, fc_b, dec_in_w, dec_in_b,
                               dec_w, dec_b, dec_g, dec_beta)


# speedup vs baseline: 1.0524x; 1.0524x over previous
import functools

import jax
import jax.numpy as jnp
from jax.experimental import pallas as pl
from jax.experimental.pallas import tpu as pltpu

_T = 256
_C = 128
_L = 128
_F = 6
_ENC_DIL = (1, 2, 4, 8)
_DEC_DIL = (8, 4, 2, 1, 1)
_ENC_LN = (True, True, True, True)
_DEC_LN = (True, True, True, True, False)
_LN_EPS = 1e-5
_SLOPE = 0.1
_MAXD = 16
_FLAT = _T * _C


def _conv_layers(buf, w_ref, b_ref, g_ref, beta_ref, bh, dilations, apply_ln,
                 ln_waits=None):
    nl = len(dilations)
    nh = 2 if bh % 2 == 0 else 1
    hh = bh // nh
    ln_idx = 0
    ys = [None] * nh
    for l in range(nl):
        d = dilations[l]
        for h in range(nh):
            b0 = h * hh
            y = None
            for tap in range(3):
                base = _MAXD + (tap - 1) * d
                lhs = buf[b0:b0 + hh, base:base + _T, :].reshape(hh * _T, _C)
                p = jnp.dot(lhs, w_ref[l, tap * _C:(tap + 1) * _C, :],
                            preferred_element_type=jnp.float32)
                y = p if y is None else y + p
            ys[h] = y.reshape(hh, _T, _C) + b_ref[l]
        if apply_ln[l] and ln_waits is not None and ln_idx in ln_waits:
            for cp in ln_waits[ln_idx]:
                cp.wait()
        for h in range(nh):
            y = ys[h]
            if apply_ln[l]:
                y = jnp.maximum(y, _SLOPE * y)
                mu = jnp.mean(y, axis=(1, 2), keepdims=True)
                msq = jnp.mean(y * y, axis=(1, 2), keepdims=True)
                var = jnp.maximum(msq - mu * mu, 0.0)
                y = (y - mu) * jax.lax.rsqrt(var + _LN_EPS)
                y = y * g_ref[ln_idx] + beta_ref[ln_idx]
                ys[h] = y
            if l + 1 < nl:
                buf[h * hh:h * hh + hh, _MAXD:_MAXD + _T, :] = \
                    y.astype(jnp.bfloat16)
        if apply_ln[l]:
            ln_idx += 1
    return ys[0] if nh == 1 else jnp.concatenate(ys, axis=0)


def _mega_body(x_ref, eps_ref, enc_w_ref, enc_b_ref, enc_g_hbm, enc_beta_hbm,
               fc_w_hbm, fc_b_ref, dec_in_w_hbm, dec_in_b_ref,
               dec_w_hbm, dec_b_ref, dec_g_hbm, dec_beta_hbm,
               recon_ref, mean_ref, logvar_ref,
               buf, enc_g_v, enc_beta_v, fc_w_v, dec_in_w_v, dec_w_v,
               dec_g_v, dec_beta_v, sems,
               *, bh):
    cp_eg = pltpu.make_async_copy(enc_g_hbm, enc_g_v, sems.at[0])
    cp_eb = pltpu.make_async_copy(enc_beta_hbm, enc_beta_v, sems.at[1])
    cp_fc = pltpu.make_async_copy(fc_w_hbm, fc_w_v, sems.at[2])
    cp_di = pltpu.make_async_copy(dec_in_w_hbm, dec_in_w_v, sems.at[3])
    cp_dw = pltpu.make_async_copy(dec_w_hbm, dec_w_v, sems.at[4])
    cp_dg = pltpu.make_async_copy(dec_g_hbm, dec_g_v, sems.at[5])
    cp_db = pltpu.make_async_copy(dec_beta_hbm, dec_beta_v, sems.at[6])
    cp_eg.start()
    cp_eb.start()
    cp_fc.start()
    cp_di.start()
    cp_dw.start()
    cp_dg.start()
    cp_db.start()

    buf[...] = jnp.zeros(buf.shape, jnp.bfloat16)
    buf[:, _MAXD:_MAXD + _T, 0:_F] = x_ref[...].astype(jnp.bfloat16)
    y = _conv_layers(buf, enc_w_ref, enc_b_ref, enc_g_v, enc_beta_v,
                     bh, _ENC_DIL, _ENC_LN,
                     ln_waits={0: [cp_eg, cp_eb]})

    flat = y.astype(jnp.bfloat16).reshape(bh, _FLAT)
    cp_fc.wait()
    y2 = jnp.dot(flat, fc_w_v[...],
                 preferred_element_type=jnp.float32) + fc_b_ref[...]
    mean = y2[:, 0:_L]
    logvar = y2[:, _L:]
    mean_ref[...] = mean
    logvar_ref[...] = logvar

    z = (mean + eps_ref[...] * jnp.exp(0.5 * logvar)).astype(jnp.bfloat16)
    cp_di.wait()
    step = 2048
    ts = step // _C
    for j in range(0, _FLAT, step):
        hj = jnp.dot(z, dec_in_w_v[:, j:j + step],
                     preferred_element_type=jnp.float32) + \
            dec_in_b_ref[:, j:j + step]
        t0 = _MAXD + j // _C
        buf[:, t0:t0 + ts, :] = hj.astype(jnp.bfloat16).reshape(bh, ts, _C)

    cp_dw.wait()
    y = _conv_layers(buf, dec_w_v, dec_b_ref, dec_g_v, dec_beta_v,
                     bh, _DEC_DIL, _DEC_LN,
                     ln_waits={0: [cp_dg, cp_db]})
    recon_ref[...] = y[:, :, 0:_F]


def _forward_one_device(x, eps, enc_w, enc_b, enc_g, enc_beta, fc_w, fc_b,
                        dec_in_w, dec_in_b, dec_w, dec_b, dec_g, dec_beta):
    B = x.shape[0]
    body = functools.partial(_mega_body, bh=B)
    any_spec = pl.BlockSpec(memory_space=pl.ANY)
    recon, mean, logvar = pl.pallas_call(
        body,
        out_shape=(jax.ShapeDtypeStruct((B, _T, _F), jnp.float32),
                   jax.ShapeDtypeStruct((B, _L), jnp.float32),
                   jax.ShapeDtypeStruct((B, _L), jnp.float32)),
        in_specs=[
            pl.BlockSpec(x.shape, lambda: (0, 0, 0)),
            pl.BlockSpec(eps.shape, lambda: (0, 0)),
            pl.BlockSpec(enc_w.shape, lambda: (0, 0, 0)),
            pl.BlockSpec(enc_b.shape, lambda: (0, 0, 0)),
            any_spec,
            any_spec,
            any_spec,
            pl.BlockSpec(fc_b.shape, lambda: (0, 0)),
            any_spec,
            pl.BlockSpec(dec_in_b.shape, lambda: (0, 0)),
            any_spec,
            pl.BlockSpec(dec_b.shape, lambda: (0, 0, 0)),
            any_spec,
            any_spec,
        ],
        out_specs=(
            pl.BlockSpec((B, _T, _F), lambda: (0, 0, 0)),
            pl.BlockSpec((B, _L), lambda: (0, 0)),
            pl.BlockSpec((B, _L), lambda: (0, 0)),
        ),
        scratch_shapes=[
            pltpu.VMEM((B, _T + 2 * _MAXD, _C), jnp.bfloat16),
            pltpu.VMEM(enc_g.shape, enc_g.dtype),
            pltpu.VMEM(enc_beta.shape, enc_beta.dtype),
            pltpu.VMEM(fc_w.shape, fc_w.dtype),
            pltpu.VMEM(dec_in_w.shape, dec_in_w.dtype),
            pltpu.VMEM(dec_w.shape, dec_w.dtype),
            pltpu.VMEM(dec_g.shape, dec_g.dtype),
            pltpu.VMEM(dec_beta.shape, dec_beta.dtype),
            pltpu.SemaphoreType.DMA((7,)),
        ],
    )(x, eps, enc_w, enc_b, enc_g, enc_beta, fc_w, fc_b,
      dec_in_w, dec_in_b, dec_w, dec_b, dec_g, dec_beta)
    return recon, mean, logvar


def kernel(x, eps, enc_w, enc_b, enc_g, enc_beta, fc_w, fc_b,
           dec_in_w, dec_in_b, dec_w, dec_b, dec_g, dec_beta):
    return _forward_one_device(x, eps, enc_w, enc_b, enc_g, enc_beta,
                               fc_w, fc_b, dec_in_w, dec_in_b,
                               dec_w, dec_b, dec_g, dec_beta)
